# baseline (speedup 1.0000x reference)
import functools

import jax
import jax.numpy as jnp
from jax import lax
from jax.experimental import pallas as pl
from jax.experimental.pallas import tpu as pltpu
from jax.experimental.pallas import tpu_sc as plsc

NC, NS, LANES = 2, 16, 16
NW = NC * NS

B, TL, AL, D = 4096, 200, 20, 64
BPW = B // NW
DC = D // LANES
RB = 2
NBLK = BPW // RB
TUNROLL = 8


def _sc_features(tflat, aflat, table, atable):
  mesh = plsc.VectorSubcoreMesh(
      core_axis_name="c", subcore_axis_name="s", num_cores=NC, num_subcores=NS)

  @functools.partial(
      pl.kernel,
      out_type=jax.ShapeDtypeStruct((B, 2 * D), jnp.float32),
      mesh=mesh,
      scratch_types=[
          pltpu.VMEM((BPW * TL,), jnp.int32),
          pltpu.VMEM((BPW * AL + LANES,), jnp.int32),
          pltpu.VMEM((RB * TL, D), jnp.float32),
          pltpu.VMEM((RB * TL, D), jnp.float32),
          pltpu.VMEM((RB * AL, D), jnp.float32),
          pltpu.VMEM((RB * AL, D), jnp.float32),
          pltpu.VMEM((BPW, 2 * D), jnp.float32),
          pltpu.SemaphoreType.DMA,
          pltpu.SemaphoreType.DMA,
      ],
      compiler_params=pltpu.CompilerParams(
          use_tc_tiling_on_sc=False, needs_layout_passes=False),
  )
  def k(tidx_hbm, aidx_hbm, tab_hbm, atab_hbm, out_hbm,
        idxt, idxa, rt0, rt1, ra0, ra1, outb, sem0, sem1):
    wid = lax.axis_index("s") * NC + lax.axis_index("c")
    base = wid * BPW
    zi = jnp.zeros((LANES,), jnp.int32)
    zf = jnp.zeros((LANES,), jnp.float32)
    lane = lax.iota(jnp.int32, LANES)

    pltpu.sync_copy(tidx_hbm.at[pl.ds(base * TL, BPW * TL)], idxt)
    pltpu.sync_copy(aidx_hbm.at[pl.ds(base * AL, BPW * AL)],
                    idxa.at[pl.ds(0, BPW * AL)])

    def issue(blk, rt, ra, sem):
      ot = pl.multiple_of(blk * (RB * TL), RB * TL)
      oa = pl.multiple_of(blk * (RB * AL), RB * AL)
      pltpu.async_copy(tab_hbm.at[idxt.at[pl.ds(ot, RB * TL)]], rt, sem)
      pltpu.async_copy(atab_hbm.at[idxa.at[pl.ds(oa, RB * AL)]], ra, sem)

    def drain(rt, ra, sem):
      pltpu.make_async_copy(
          tab_hbm.at[idxt.at[pl.ds(0, RB * TL)]], rt, sem).wait()
      pltpu.make_async_copy(
          atab_hbm.at[idxa.at[pl.ds(0, RB * AL)]], ra, sem).wait()

    def nzcount(v, m=None):
      nz = v != 0
      if m is not None:
        nz = jnp.logical_and(nz, m)
      return plsc.all_reduce_population_count(nz)

    def compute(blk, rt, ra):
      oa = pl.multiple_of(blk * (RB * AL), RB * AL)
      a0 = idxa[pl.ds(oa, LANES)]
      a1 = idxa[pl.ds(oa + 16, LANES)]
      a2 = idxa[pl.ds(oa + 32, LANES)]
      la_r = (nzcount(a0) + nzcount(a1, lane < 4),
              nzcount(a1, lane >= 4) + nzcount(a2, lane < 8))

      for r in range(RB):
        b = blk * RB + r
        ot = pl.multiple_of(b * TL, TL)
        lt = zi
        for c in range(TL // LANES):
          lt = lt + nzcount(idxt[pl.ds(ot + c * LANES, LANES)])
        lt = lt + nzcount(idxt[pl.ds(ot + TL - LANES, LANES)], lane >= 8)
        la = la_r[r]

        def tstep(j, accs):
          accs = list(accs)
          for u in range(TUNROLL):
            p = j * TUNROLL + u
            m = lt > p
            for d in range(DC):
              v = rt[r * TL + p, pl.ds(d * LANES, LANES)]
              accs[d] = accs[d] + jnp.where(m, v, 0.0)
          return tuple(accs)
        acc_t = list(lax.fori_loop(0, TL // TUNROLL, tstep, (zf,) * DC))

        acc_a = [zf] * DC
        for p in range(AL):
          m = la > p
          for d in range(DC):
            v = ra[r * AL + p, pl.ds(d * LANES, LANES)]
            acc_a[d] = acc_a[d] + jnp.where(m, v, 0.0)

        inv_t = 1.0 / jnp.maximum(lt.astype(jnp.float32), 1.0)
        inv_a = 1.0 / jnp.maximum(la.astype(jnp.float32), 1.0)
        for d in range(DC):
          outb[b, pl.ds(d * LANES, LANES)] = acc_t[d] * inv_t
          outb[b, pl.ds(D + d * LANES, LANES)] = acc_a[d] * inv_a

    issue(0, rt0, ra0, sem0)

    def body(i, carry):
      blk = 2 * i
      drain(rt0, ra0, sem0)
      issue(blk + 1, rt1, ra1, sem1)
      compute(blk, rt0, ra0)
      drain(rt1, ra1, sem1)

      @pl.when(blk + 2 < NBLK)
      def _():
        issue(blk + 2, rt0, ra0, sem0)
      compute(blk + 1, rt1, ra1)
      return carry

    lax.fori_loop(0, NBLK // 2, body, 0)
    pltpu.sync_copy(outb, out_hbm.at[pl.ds(base, BPW)])

  return k(tflat, aflat, table, atable)


TCH = 2048


def _tc_transpose(t_view):
  v = t_view.shape[1]

  def body(in_ref, out_ref):
    out_ref[...] = in_ref[...].T

  return pl.pallas_call(
      body,
      grid=((v + TCH - 1) // TCH,),
      in_specs=[pl.BlockSpec((D, TCH), lambda g: (0, g))],
      out_specs=pl.BlockSpec((TCH, D), lambda g: (g, 0)),
      out_shape=jax.ShapeDtypeStruct((v, D), jnp.float32),
  )(t_view)


def _tc_logits(feat, w, bias):
  def body(f_ref, w_ref, b_ref, o_ref):
    o_ref[...] = lax.dot_general(
        f_ref[...], w_ref[...], (((1,), (1,)), ((), ())),
        preferred_element_type=jnp.float32,
        precision=lax.Precision.HIGHEST) + b_ref[...]

  return pl.pallas_call(
      body,
      out_shape=jax.ShapeDtypeStruct((B, w.shape[0]), jnp.float32),
  )(feat, w, bias.reshape(1, -1))


def kernel(text_raw_indices, aspect_indices, embedding_matrix,
           aspect_embedding_matrix, W, b):
  tflat = text_raw_indices.astype(jnp.int32).reshape(-1)
  aflat = aspect_indices.astype(jnp.int32).reshape(-1)
  table = _tc_transpose(embedding_matrix.T)
  atable = _tc_transpose(aspect_embedding_matrix.T)
  feat = _sc_features(tflat, aflat, table, atable)
  return _tc_logits(feat, W, b)


# speedup vs baseline: 1.2774x; 1.2774x over previous
import functools

import jax
import jax.numpy as jnp
from jax import lax
from jax.experimental import pallas as pl
from jax.experimental.pallas import tpu as pltpu
from jax.experimental.pallas import tpu_sc as plsc

NC, NS, LANES = 2, 16, 16
NW = NC * NS

B, TL, AL, D = 4096, 200, 20, 64
BPW = B // NW
DC = D // LANES
RB = 2
NBLK = BPW // RB
TUNROLL = 8


def _sc_features(tflat, aflat, table, atable):
  mesh = plsc.VectorSubcoreMesh(
      core_axis_name="c", subcore_axis_name="s", num_cores=NC, num_subcores=NS)

  @functools.partial(
      pl.kernel,
      out_type=jax.ShapeDtypeStruct((B, 2 * D), jnp.float32),
      mesh=mesh,
      scratch_types=[
          pltpu.VMEM((BPW * TL,), jnp.int32),
          pltpu.VMEM((BPW * AL + LANES,), jnp.int32),
          pltpu.VMEM((RB * TL, D), jnp.float32),
          pltpu.VMEM((RB * TL, D), jnp.float32),
          pltpu.VMEM((RB * AL, D), jnp.float32),
          pltpu.VMEM((RB * AL, D), jnp.float32),
          pltpu.VMEM((BPW, 2 * D), jnp.float32),
          pltpu.SemaphoreType.DMA,
          pltpu.SemaphoreType.DMA,
      ],
      compiler_params=pltpu.CompilerParams(
          use_tc_tiling_on_sc=False, needs_layout_passes=False),
  )
  def k(tidx_hbm, aidx_hbm, tab_hbm, atab_hbm, out_hbm,
        idxt, idxa, rt0, rt1, ra0, ra1, outb, sem0, sem1):
    wid = lax.axis_index("s") * NC + lax.axis_index("c")
    base = wid * BPW
    zi = jnp.zeros((LANES,), jnp.int32)
    zf = jnp.zeros((LANES,), jnp.float32)
    lane = lax.iota(jnp.int32, LANES)

    pltpu.sync_copy(tidx_hbm.at[pl.ds(base * TL, BPW * TL)], idxt)
    pltpu.sync_copy(aidx_hbm.at[pl.ds(base * AL, BPW * AL)],
                    idxa.at[pl.ds(0, BPW * AL)])

    def issue(blk, rt, ra, sem):
      ot = pl.multiple_of(blk * (RB * TL), RB * TL)
      oa = pl.multiple_of(blk * (RB * AL), RB * AL)
      pltpu.async_copy(tab_hbm.at[idxt.at[pl.ds(ot, RB * TL)]], rt, sem)
      pltpu.async_copy(atab_hbm.at[idxa.at[pl.ds(oa, RB * AL)]], ra, sem)

    def drain(rt, ra, sem):
      pltpu.make_async_copy(
          tab_hbm.at[idxt.at[pl.ds(0, RB * TL)]], rt, sem).wait()
      pltpu.make_async_copy(
          atab_hbm.at[idxa.at[pl.ds(0, RB * AL)]], ra, sem).wait()

    def nzcount(v, m=None):
      nz = v != 0
      if m is not None:
        nz = jnp.logical_and(nz, m)
      return plsc.all_reduce_population_count(nz)

    def compute(blk, rt, ra):
      oa = pl.multiple_of(blk * (RB * AL), RB * AL)
      a0 = idxa[pl.ds(oa, LANES)]
      a1 = idxa[pl.ds(oa + 16, LANES)]
      a2 = idxa[pl.ds(oa + 32, LANES)]
      la_r = (nzcount(a0) + nzcount(a1, lane < 4),
              nzcount(a1, lane >= 4) + nzcount(a2, lane < 8))

      for r in range(RB):
        b = blk * RB + r
        ot = pl.multiple_of(b * TL, TL)
        lt = zi
        for c in range(TL // LANES):
          lt = lt + nzcount(idxt[pl.ds(ot + c * LANES, LANES)])
        lt = lt + nzcount(idxt[pl.ds(ot + TL - LANES, LANES)], lane >= 8)
        la = la_r[r]

        def tstep(j, accs):
          accs = list(accs)
          for u in range(TUNROLL):
            p = j * TUNROLL + u
            m = lt > p
            for d in range(DC):
              v = rt[r * TL + p, pl.ds(d * LANES, LANES)]
              accs[d] = accs[d] + jnp.where(m, v, 0.0)
          return tuple(accs)
        acc_t = list(lax.fori_loop(0, TL // TUNROLL, tstep, (zf,) * DC))

        acc_a = [zf] * DC
        for p in range(AL):
          m = la > p
          for d in range(DC):
            v = ra[r * AL + p, pl.ds(d * LANES, LANES)]
            acc_a[d] = acc_a[d] + jnp.where(m, v, 0.0)

        inv_t = 1.0 / jnp.maximum(lt.astype(jnp.float32), 1.0)
        inv_a = 1.0 / jnp.maximum(la.astype(jnp.float32), 1.0)
        for d in range(DC):
          outb[b, pl.ds(d * LANES, LANES)] = acc_t[d] * inv_t
          outb[b, pl.ds(D + d * LANES, LANES)] = acc_a[d] * inv_a

    issue(0, rt0, ra0, sem0)

    def body(i, carry):
      blk = 2 * i
      drain(rt0, ra0, sem0)
      issue(blk + 1, rt1, ra1, sem1)
      compute(blk, rt0, ra0)
      drain(rt1, ra1, sem1)

      @pl.when(blk + 2 < NBLK)
      def _():
        issue(blk + 2, rt0, ra0, sem0)
      compute(blk + 1, rt1, ra1)
      return carry

    lax.fori_loop(0, NBLK // 2, body, 0)
    pltpu.sync_copy(outb, out_hbm.at[pl.ds(base, BPW)])

  return k(tflat, aflat, table, atable)


TCH = 8192


def _tc_transpose(t_view):
  v = t_view.shape[1]

  def body(in_ref, out_ref):
    out_ref[...] = in_ref[...].T

  return pl.pallas_call(
      body,
      grid=((v + TCH - 1) // TCH,),
      in_specs=[pl.BlockSpec((D, TCH), lambda g: (0, g))],
      out_specs=pl.BlockSpec((TCH, D), lambda g: (g, 0)),
      out_shape=jax.ShapeDtypeStruct((v, D), jnp.float32),
  )(t_view)


def _tc_logits(feat, w, bias):
  def body(f_ref, w_ref, b_ref, o_ref):
    o_ref[...] = lax.dot_general(
        f_ref[...], w_ref[...], (((1,), (1,)), ((), ())),
        preferred_element_type=jnp.float32,
        precision=lax.Precision.HIGHEST) + b_ref[...]

  return pl.pallas_call(
      body,
      out_shape=jax.ShapeDtypeStruct((B, w.shape[0]), jnp.float32),
  )(feat, w, bias.reshape(1, -1))


def kernel(text_raw_indices, aspect_indices, embedding_matrix,
           aspect_embedding_matrix, W, b):
  tflat = text_raw_indices.astype(jnp.int32).reshape(-1)
  aflat = aspect_indices.astype(jnp.int32).reshape(-1)
  table = _tc_transpose(embedding_matrix.T)
  atable = _tc_transpose(aspect_embedding_matrix.T)
  feat = _sc_features(tflat, aflat, table, atable)
  return _tc_logits(feat, W, b)


# speedup vs baseline: 1.3775x; 1.0784x over previous
import functools

import jax
import jax.numpy as jnp
from jax import lax
from jax.experimental import pallas as pl
from jax.experimental.pallas import tpu as pltpu
from jax.experimental.pallas import tpu_sc as plsc

NC, NS, LANES = 2, 16, 16
NW = NC * NS

B, TL, AL, D = 4096, 200, 20, 64
BPW = B // NW
DC = D // LANES
RB = 2
NBLK = BPW // RB
TUNROLL = 8


def _sc_features(tflat, aflat, table, atable):
  mesh = plsc.VectorSubcoreMesh(
      core_axis_name="c", subcore_axis_name="s", num_cores=NC, num_subcores=NS)

  @functools.partial(
      pl.kernel,
      out_type=jax.ShapeDtypeStruct((B, 2 * D), jnp.float32),
      mesh=mesh,
      scratch_types=[
          pltpu.VMEM((BPW * TL,), jnp.int32),
          pltpu.VMEM((BPW * AL + LANES,), jnp.int32),
          pltpu.VMEM((RB * TL, D), jnp.float32),
          pltpu.VMEM((RB * TL, D), jnp.float32),
          pltpu.VMEM((RB * AL, D), jnp.float32),
          pltpu.VMEM((RB * AL, D), jnp.float32),
          pltpu.VMEM((BPW, 2 * D), jnp.float32),
          pltpu.SemaphoreType.DMA,
          pltpu.SemaphoreType.DMA,
      ],
      compiler_params=pltpu.CompilerParams(
          use_tc_tiling_on_sc=False, needs_layout_passes=False),
  )
  def k(tidx_hbm, aidx_hbm, tab_hbm, atab_hbm, out_hbm,
        idxt, idxa, rt0, rt1, ra0, ra1, outb, sem0, sem1):
    wid = lax.axis_index("s") * NC + lax.axis_index("c")
    base = wid * BPW
    zi = jnp.zeros((LANES,), jnp.int32)
    zf = jnp.zeros((LANES,), jnp.float32)
    lane = lax.iota(jnp.int32, LANES)

    pltpu.sync_copy(tidx_hbm.at[pl.ds(base * TL, BPW * TL)], idxt)
    pltpu.sync_copy(aidx_hbm.at[pl.ds(base * AL, BPW * AL)],
                    idxa.at[pl.ds(0, BPW * AL)])

    def issue(blk, rt, ra, sem):
      ot = pl.multiple_of(blk * (RB * TL), RB * TL)
      oa = pl.multiple_of(blk * (RB * AL), RB * AL)
      pltpu.async_copy(tab_hbm.at[idxt.at[pl.ds(ot, RB * TL)]], rt, sem)
      pltpu.async_copy(atab_hbm.at[idxa.at[pl.ds(oa, RB * AL)]], ra, sem)

    def drain(rt, ra, sem):
      pltpu.make_async_copy(
          tab_hbm.at[idxt.at[pl.ds(0, RB * TL)]], rt, sem).wait()
      pltpu.make_async_copy(
          atab_hbm.at[idxa.at[pl.ds(0, RB * AL)]], ra, sem).wait()

    def nzcount(v, m=None):
      nz = v != 0
      if m is not None:
        nz = jnp.logical_and(nz, m)
      return plsc.all_reduce_population_count(nz)

    def compute(blk, rt, ra):
      oa = pl.multiple_of(blk * (RB * AL), RB * AL)
      a0 = idxa[pl.ds(oa, LANES)]
      a1 = idxa[pl.ds(oa + 16, LANES)]
      a2 = idxa[pl.ds(oa + 32, LANES)]
      la_r = (nzcount(a0) + nzcount(a1, lane < 4),
              nzcount(a1, lane >= 4) + nzcount(a2, lane < 8))

      for r in range(RB):
        b = blk * RB + r
        ot = pl.multiple_of(b * TL, TL)
        lt = zi
        for c in range(TL // LANES):
          lt = lt + nzcount(idxt[pl.ds(ot + c * LANES, LANES)])
        lt = lt + nzcount(idxt[pl.ds(ot + TL - LANES, LANES)], lane >= 8)
        la = la_r[r]

        def tstep(j, accs):
          accs = list(accs)
          for u in range(TUNROLL):
            p = j * TUNROLL + u
            m = lt > p
            for d in range(DC):
              v = rt[r * TL + p, pl.ds(d * LANES, LANES)]
              accs[d] = accs[d] + jnp.where(m, v, 0.0)
          return tuple(accs)
        acc_t = list(lax.fori_loop(0, TL // TUNROLL, tstep, (zf,) * DC))

        acc_a = [zf] * DC
        for p in range(AL):
          m = la > p
          for d in range(DC):
            v = ra[r * AL + p, pl.ds(d * LANES, LANES)]
            acc_a[d] = acc_a[d] + jnp.where(m, v, 0.0)

        inv_t = 1.0 / jnp.maximum(lt.astype(jnp.float32), 1.0)
        inv_a = 1.0 / jnp.maximum(la.astype(jnp.float32), 1.0)
        for d in range(DC):
          outb[b, pl.ds(d * LANES, LANES)] = acc_t[d] * inv_t
          outb[b, pl.ds(D + d * LANES, LANES)] = acc_a[d] * inv_a

    issue(0, rt0, ra0, sem0)

    def body(i, carry):
      blk = 2 * i
      drain(rt0, ra0, sem0)
      issue(blk + 1, rt1, ra1, sem1)
      compute(blk, rt0, ra0)
      drain(rt1, ra1, sem1)

      @pl.when(blk + 2 < NBLK)
      def _():
        issue(blk + 2, rt0, ra0, sem0)
      compute(blk + 1, rt1, ra1)
      return carry

    lax.fori_loop(0, NBLK // 2, body, 0)
    pltpu.sync_copy(outb, out_hbm.at[pl.ds(base, BPW)])

  return k(tflat, aflat, table, atable)


TCH = 16384


def _tc_transpose(t_view):
  v = t_view.shape[1]

  def body(in_ref, out_ref):
    out_ref[...] = in_ref[...].T

  return pl.pallas_call(
      body,
      grid=((v + TCH - 1) // TCH,),
      in_specs=[pl.BlockSpec((D, TCH), lambda g: (0, g))],
      out_specs=pl.BlockSpec((TCH, D), lambda g: (g, 0)),
      out_shape=jax.ShapeDtypeStruct((v, D), jnp.float32),
  )(t_view)


def _tc_logits(feat, w, bias):
  def body(f_ref, w_ref, b_ref, o_ref):
    o_ref[...] = lax.dot_general(
        f_ref[...], w_ref[...], (((1,), (1,)), ((), ())),
        preferred_element_type=jnp.float32,
        precision=lax.Precision.HIGHEST) + b_ref[...]

  return pl.pallas_call(
      body,
      out_shape=jax.ShapeDtypeStruct((B, w.shape[0]), jnp.float32),
  )(feat, w, bias.reshape(1, -1))


def kernel(text_raw_indices, aspect_indices, embedding_matrix,
           aspect_embedding_matrix, W, b):
  tflat = text_raw_indices.astype(jnp.int32).reshape(-1)
  aflat = aspect_indices.astype(jnp.int32).reshape(-1)
  atable = _tc_transpose(aspect_embedding_matrix.T)
  feat = _sc_features(tflat, aflat, embedding_matrix, atable)
  return _tc_logits(feat, W, b)


# speedup vs baseline: 1.8662x; 1.3547x over previous
import functools

import jax
import jax.numpy as jnp
from jax import lax
from jax.experimental import pallas as pl
from jax.experimental.pallas import tpu as pltpu
from jax.experimental.pallas import tpu_sc as plsc

NC, NS, LANES = 2, 16, 16
NW = NC * NS

B, TL, AL, D = 4096, 200, 20, 64
BPW = B // NW
DC = D // LANES
RB = 2
NBLK = BPW // RB
TUNROLL = 8


def _sc_features(tflat, aflat, table, atable):
  mesh = plsc.VectorSubcoreMesh(
      core_axis_name="c", subcore_axis_name="s", num_cores=NC, num_subcores=NS)

  @functools.partial(
      pl.kernel,
      out_type=jax.ShapeDtypeStruct((B, 2 * D), jnp.float32),
      mesh=mesh,
      scratch_types=[
          pltpu.VMEM((BPW * TL,), jnp.int32),
          pltpu.VMEM((BPW * AL + LANES,), jnp.int32),
          pltpu.VMEM((RB * TL, D), jnp.float32),
          pltpu.VMEM((RB * TL, D), jnp.float32),
          pltpu.VMEM((RB * AL, D), jnp.float32),
          pltpu.VMEM((RB * AL, D), jnp.float32),
          pltpu.VMEM((BPW, 2 * D), jnp.float32),
          pltpu.SemaphoreType.DMA,
          pltpu.SemaphoreType.DMA,
      ],
      compiler_params=pltpu.CompilerParams(
          use_tc_tiling_on_sc=False, needs_layout_passes=False),
  )
  def k(tidx_hbm, aidx_hbm, tab_hbm, atab_hbm, out_hbm,
        idxt, idxa, rt0, rt1, ra0, ra1, outb, sem0, sem1):
    wid = lax.axis_index("s") * NC + lax.axis_index("c")
    base = wid * BPW
    zi = jnp.zeros((LANES,), jnp.int32)
    zf = jnp.zeros((LANES,), jnp.float32)
    lane = lax.iota(jnp.int32, LANES)

    pltpu.sync_copy(tidx_hbm.at[pl.ds(base * TL, BPW * TL)], idxt)
    pltpu.sync_copy(aidx_hbm.at[pl.ds(base * AL, BPW * AL)],
                    idxa.at[pl.ds(0, BPW * AL)])

    def issue(blk, rt, ra, sem):
      ot = pl.multiple_of(blk * (RB * TL), RB * TL)
      oa = pl.multiple_of(blk * (RB * AL), RB * AL)
      pltpu.async_copy(tab_hbm.at[idxt.at[pl.ds(ot, RB * TL)]], rt, sem)
      pltpu.async_copy(atab_hbm.at[idxa.at[pl.ds(oa, RB * AL)]], ra, sem)

    def drain(rt, ra, sem):
      pltpu.make_async_copy(
          tab_hbm.at[idxt.at[pl.ds(0, RB * TL)]], rt, sem).wait()
      pltpu.make_async_copy(
          atab_hbm.at[idxa.at[pl.ds(0, RB * AL)]], ra, sem).wait()

    def nzcount(v, m=None):
      nz = v != 0
      if m is not None:
        nz = jnp.logical_and(nz, m)
      return plsc.all_reduce_population_count(nz)

    def compute(blk, rt, ra):
      oa = pl.multiple_of(blk * (RB * AL), RB * AL)
      a0 = idxa[pl.ds(oa, LANES)]
      a1 = idxa[pl.ds(oa + 16, LANES)]
      a2 = idxa[pl.ds(oa + 32, LANES)]
      la_r = (nzcount(a0) + nzcount(a1, lane < 4),
              nzcount(a1, lane >= 4) + nzcount(a2, lane < 8))

      for r in range(RB):
        b = blk * RB + r
        ot = pl.multiple_of(b * TL, TL)
        lt = zi
        for c in range(TL // LANES):
          lt = lt + nzcount(idxt[pl.ds(ot + c * LANES, LANES)])
        lt = lt + nzcount(idxt[pl.ds(ot + TL - LANES, LANES)], lane >= 8)
        la = la_r[r]

        def tstep(j, accs):
          accs = list(accs)
          for u in range(TUNROLL):
            p = j * TUNROLL + u
            m = lt > p
            for d in range(DC):
              v = rt[r * TL + p, pl.ds(d * LANES, LANES)]
              accs[d] = accs[d] + jnp.where(m, v, 0.0)
          return tuple(accs)
        acc_t = list(lax.fori_loop(0, TL // TUNROLL, tstep, (zf,) * DC))

        acc_a = [zf] * DC
        for p in range(AL):
          m = la > p
          for d in range(DC):
            v = ra[r * AL + p, pl.ds(d * LANES, LANES)]
            acc_a[d] = acc_a[d] + jnp.where(m, v, 0.0)

        inv_t = 1.0 / jnp.maximum(lt.astype(jnp.float32), 1.0)
        inv_a = 1.0 / jnp.maximum(la.astype(jnp.float32), 1.0)
        for d in range(DC):
          outb[b, pl.ds(d * LANES, LANES)] = acc_t[d] * inv_t
          outb[b, pl.ds(D + d * LANES, LANES)] = acc_a[d] * inv_a

    issue(0, rt0, ra0, sem0)

    def body(i, carry):
      blk = 2 * i
      drain(rt0, ra0, sem0)
      issue(blk + 1, rt1, ra1, sem1)
      compute(blk, rt0, ra0)
      drain(rt1, ra1, sem1)

      @pl.when(blk + 2 < NBLK)
      def _():
        issue(blk + 2, rt0, ra0, sem0)
      compute(blk + 1, rt1, ra1)
      return carry

    lax.fori_loop(0, NBLK // 2, body, 0)
    pltpu.sync_copy(outb, out_hbm.at[pl.ds(base, BPW)])

  return k(tflat, aflat, table, atable)


TCH = 4096


def _tc_transpose(t_view):
  v = t_view.shape[1]

  def body(in_ref, out_ref):
    a = in_ref[...].T.reshape(TCH // 2, 2, D)
    out_ref[...] = jnp.concatenate([a[:, 0, :], a[:, 1, :]], axis=-1)

  out = pl.pallas_call(
      body,
      grid=((v + TCH - 1) // TCH,),
      in_specs=[pl.BlockSpec((D, TCH), lambda g: (0, g))],
      out_specs=pl.BlockSpec((TCH // 2, 2 * D), lambda g: (g, 0)),
      out_shape=jax.ShapeDtypeStruct((v // 2, 2 * D), jnp.float32),
  )(t_view)
  return out.reshape(v, D)


def _tc_logits(feat, w, bias):
  def body(f_ref, w_ref, b_ref, o_ref):
    o_ref[...] = lax.dot_general(
        f_ref[...], w_ref[...], (((1,), (1,)), ((), ())),
        preferred_element_type=jnp.float32,
        precision=lax.Precision.HIGHEST) + b_ref[...]

  return pl.pallas_call(
      body,
      out_shape=jax.ShapeDtypeStruct((B, w.shape[0]), jnp.float32),
  )(feat, w, bias.reshape(1, -1))


def kernel(text_raw_indices, aspect_indices, embedding_matrix,
           aspect_embedding_matrix, W, b):
  tflat = text_raw_indices.astype(jnp.int32).reshape(-1)
  aflat = aspect_indices.astype(jnp.int32).reshape(-1)
  table = _tc_transpose(embedding_matrix.T)
  atable = _tc_transpose(aspect_embedding_matrix.T)
  feat = _sc_features(tflat, aflat, table, atable)
  return _tc_logits(feat, W, b)


# speedup vs baseline: 2.2775x; 1.2204x over previous
import functools

import jax
import jax.numpy as jnp
from jax import lax
from jax.experimental import pallas as pl
from jax.experimental.pallas import tpu as pltpu
from jax.experimental.pallas import tpu_sc as plsc

NC, NS, LANES = 2, 16, 16
NW = NC * NS

B, TL, AL, D = 4096, 200, 20, 64
BPW = B // NW
DC = D // LANES
RB = 2
NBLK = BPW // RB
TUNROLL = 8


def _sc_features(tflat, aflat, table, atable):
  mesh = plsc.VectorSubcoreMesh(
      core_axis_name="c", subcore_axis_name="s", num_cores=NC, num_subcores=NS)

  @functools.partial(
      pl.kernel,
      out_type=jax.ShapeDtypeStruct((B, 2 * D), jnp.float32),
      mesh=mesh,
      scratch_types=[
          pltpu.VMEM((BPW * TL,), jnp.int32),
          pltpu.VMEM((BPW * AL + LANES,), jnp.int32),
          pltpu.VMEM((RB * TL, D), jnp.float32),
          pltpu.VMEM((RB * TL, D), jnp.float32),
          pltpu.VMEM((RB * AL, D), jnp.float32),
          pltpu.VMEM((RB * AL, D), jnp.float32),
          pltpu.VMEM((BPW, 2 * D), jnp.float32),
          pltpu.SemaphoreType.DMA,
          pltpu.SemaphoreType.DMA,
      ],
      compiler_params=pltpu.CompilerParams(
          use_tc_tiling_on_sc=False, needs_layout_passes=False),
  )
  def k(tidx_hbm, aidx_hbm, tab_hbm, atab_hbm, out_hbm,
        idxt, idxa, rt0, rt1, ra0, ra1, outb, sem0, sem1):
    wid = lax.axis_index("s") * NC + lax.axis_index("c")
    base = wid * BPW
    zi = jnp.zeros((LANES,), jnp.int32)
    zf = jnp.zeros((LANES,), jnp.float32)
    lane = lax.iota(jnp.int32, LANES)

    pltpu.sync_copy(tidx_hbm.at[pl.ds(base * TL, BPW * TL)], idxt)
    pltpu.sync_copy(aidx_hbm.at[pl.ds(base * AL, BPW * AL)],
                    idxa.at[pl.ds(0, BPW * AL)])

    def _remap(v):
      return ((v & jnp.int32(-TCH)) | ((v & (TCH // 2 - 1)) << 1)
              | ((v >> TSH) & 1))

    def _rloop(buf, nwords):
      def st(j, c):
        off = pl.multiple_of(j * LANES, LANES)
        buf[pl.ds(off, LANES)] = _remap(buf[pl.ds(off, LANES)])
        return c
      lax.fori_loop(0, nwords // LANES, st, 0)

    _rloop(idxt, BPW * TL)
    _rloop(idxa, BPW * AL)

    def issue(blk, rt, ra, sem):
      ot = pl.multiple_of(blk * (RB * TL), RB * TL)
      oa = pl.multiple_of(blk * (RB * AL), RB * AL)
      pltpu.async_copy(tab_hbm.at[idxt.at[pl.ds(ot, RB * TL)]], rt, sem)
      pltpu.async_copy(atab_hbm.at[idxa.at[pl.ds(oa, RB * AL)]], ra, sem)

    def drain(rt, ra, sem):
      pltpu.make_async_copy(
          tab_hbm.at[idxt.at[pl.ds(0, RB * TL)]], rt, sem).wait()
      pltpu.make_async_copy(
          atab_hbm.at[idxa.at[pl.ds(0, RB * AL)]], ra, sem).wait()

    def nzcount(v, m=None):
      nz = v != 0
      if m is not None:
        nz = jnp.logical_and(nz, m)
      return plsc.all_reduce_population_count(nz)

    def compute(blk, rt, ra):
      oa = pl.multiple_of(blk * (RB * AL), RB * AL)
      a0 = idxa[pl.ds(oa, LANES)]
      a1 = idxa[pl.ds(oa + 16, LANES)]
      a2 = idxa[pl.ds(oa + 32, LANES)]
      la_r = (nzcount(a0) + nzcount(a1, lane < 4),
              nzcount(a1, lane >= 4) + nzcount(a2, lane < 8))

      for r in range(RB):
        b = blk * RB + r
        ot = pl.multiple_of(b * TL, TL)
        lt = zi
        for c in range(TL // LANES):
          lt = lt + nzcount(idxt[pl.ds(ot + c * LANES, LANES)])
        lt = lt + nzcount(idxt[pl.ds(ot + TL - LANES, LANES)], lane >= 8)
        la = la_r[r]

        def tstep(j, accs):
          accs = list(accs)
          for u in range(TUNROLL):
            p = j * TUNROLL + u
            m = lt > p
            for d in range(DC):
              v = rt[r * TL + p, pl.ds(d * LANES, LANES)]
              accs[d] = accs[d] + jnp.where(m, v, 0.0)
          return tuple(accs)
        acc_t = list(lax.fori_loop(0, TL // TUNROLL, tstep, (zf,) * DC))

        acc_a = [zf] * DC
        for p in range(AL):
          m = la > p
          for d in range(DC):
            v = ra[r * AL + p, pl.ds(d * LANES, LANES)]
            acc_a[d] = acc_a[d] + jnp.where(m, v, 0.0)

        inv_t = 1.0 / jnp.maximum(lt.astype(jnp.float32), 1.0)
        inv_a = 1.0 / jnp.maximum(la.astype(jnp.float32), 1.0)
        for d in range(DC):
          outb[b, pl.ds(d * LANES, LANES)] = acc_t[d] * inv_t
          outb[b, pl.ds(D + d * LANES, LANES)] = acc_a[d] * inv_a

    issue(0, rt0, ra0, sem0)

    def body(i, carry):
      blk = 2 * i
      drain(rt0, ra0, sem0)
      issue(blk + 1, rt1, ra1, sem1)
      compute(blk, rt0, ra0)
      drain(rt1, ra1, sem1)

      @pl.when(blk + 2 < NBLK)
      def _():
        issue(blk + 2, rt0, ra0, sem0)
      compute(blk + 1, rt1, ra1)
      return carry

    lax.fori_loop(0, NBLK // 2, body, 0)
    pltpu.sync_copy(outb, out_hbm.at[pl.ds(base, BPW)])

  return k(tflat, aflat, table, atable)


TCH = 4096
TSH = 11


def _tc_transpose(t_view):
  v = t_view.shape[1]
  nblk = (v + TCH - 1) // TCH

  def body(in_ref, out_ref):
    t = in_ref[...].T
    out_ref[...] = jnp.concatenate([t[:TCH // 2], t[TCH // 2:]], axis=-1)

  out = pl.pallas_call(
      body,
      grid=(nblk,),
      in_specs=[pl.BlockSpec((D, TCH), lambda g: (0, g))],
      out_specs=pl.BlockSpec((TCH // 2, 2 * D), lambda g: (g, 0)),
      out_shape=jax.ShapeDtypeStruct((nblk * TCH // 2, 2 * D), jnp.float32),
  )(t_view)
  return out.reshape(nblk * TCH, D)


def _tc_logits(feat, w, bias):
  def body(f_ref, w_ref, b_ref, o_ref):
    o_ref[...] = lax.dot_general(
        f_ref[...], w_ref[...], (((1,), (1,)), ((), ())),
        preferred_element_type=jnp.float32,
        precision=lax.Precision.HIGHEST) + b_ref[...]

  return pl.pallas_call(
      body,
      out_shape=jax.ShapeDtypeStruct((B, w.shape[0]), jnp.float32),
  )(feat, w, bias.reshape(1, -1))


def kernel(text_raw_indices, aspect_indices, embedding_matrix,
           aspect_embedding_matrix, W, b):
  tflat = text_raw_indices.astype(jnp.int32).reshape(-1)
  aflat = aspect_indices.astype(jnp.int32).reshape(-1)
  table = _tc_transpose(embedding_matrix.T)
  atable = _tc_transpose(aspect_embedding_matrix.T)
  feat = _sc_features(tflat, aflat, table, atable)
  return _tc_logits(feat, W, b)


# speedup vs baseline: 2.7284x; 1.1980x over previous
import functools

import jax
import jax.numpy as jnp
from jax import lax
from jax.experimental import pallas as pl
from jax.experimental.pallas import tpu as pltpu
from jax.experimental.pallas import tpu_sc as plsc

NC, NS, LANES = 2, 16, 16
NW = NC * NS

B, TL, AL, D = 4096, 200, 20, 64
BPW = B // NW
DC = D // LANES
RB = 2
NBLK = BPW // RB
TUNROLL = 8


def _sc_features(tflat, aflat, table, atable):
  mesh = plsc.VectorSubcoreMesh(
      core_axis_name="c", subcore_axis_name="s", num_cores=NC, num_subcores=NS)

  @functools.partial(
      pl.kernel,
      out_type=jax.ShapeDtypeStruct((B, 2 * D), jnp.float32),
      mesh=mesh,
      scratch_types=[
          pltpu.VMEM((BPW * TL,), jnp.int32),
          pltpu.VMEM((BPW * AL + LANES,), jnp.int32),
          pltpu.VMEM((RB * TL, D), jnp.float32),
          pltpu.VMEM((RB * TL, D), jnp.float32),
          pltpu.VMEM((RB * AL, D), jnp.float32),
          pltpu.VMEM((RB * AL, D), jnp.float32),
          pltpu.VMEM((BPW, 2 * D), jnp.float32),
          pltpu.SemaphoreType.DMA,
          pltpu.SemaphoreType.DMA,
      ],
      compiler_params=pltpu.CompilerParams(
          use_tc_tiling_on_sc=False, needs_layout_passes=False),
  )
  def k(tidx_hbm, aidx_hbm, tab_hbm, atab_hbm, out_hbm,
        idxt, idxa, rt0, rt1, ra0, ra1, outb, sem0, sem1):
    wid = lax.axis_index("s") * NC + lax.axis_index("c")
    base = wid * BPW
    zi = jnp.zeros((LANES,), jnp.int32)
    zf = jnp.zeros((LANES,), jnp.float32)
    lane = lax.iota(jnp.int32, LANES)

    pltpu.sync_copy(tidx_hbm.at[pl.ds(base * TL, BPW * TL)], idxt)
    pltpu.sync_copy(aidx_hbm.at[pl.ds(base * AL, BPW * AL)],
                    idxa.at[pl.ds(0, BPW * AL)])

    def _remap(v):
      return ((v & jnp.int32(-TCH)) | ((v & (TCH // 2 - 1)) << 1)
              | ((v >> TSH) & 1))

    def _rloop(buf, nwords):
      def st(j, c):
        off = pl.multiple_of(j * LANES, LANES)
        buf[pl.ds(off, LANES)] = _remap(buf[pl.ds(off, LANES)])
        return c
      lax.fori_loop(0, nwords // LANES, st, 0)

    _rloop(idxt, BPW * TL)
    _rloop(idxa, BPW * AL)

    def issue(blk, rt, ra, sem):
      ot = pl.multiple_of(blk * (RB * TL), RB * TL)
      oa = pl.multiple_of(blk * (RB * AL), RB * AL)
      pltpu.async_copy(tab_hbm.at[idxt.at[pl.ds(ot, RB * TL)]], rt, sem)
      pltpu.async_copy(atab_hbm.at[idxa.at[pl.ds(oa, RB * AL)]], ra, sem)

    def drain(rt, ra, sem):
      pltpu.make_async_copy(
          tab_hbm.at[idxt.at[pl.ds(0, RB * TL)]], rt, sem).wait()
      pltpu.make_async_copy(
          atab_hbm.at[idxa.at[pl.ds(0, RB * AL)]], ra, sem).wait()

    def nzcount(v, m=None):
      nz = v != 0
      if m is not None:
        nz = jnp.logical_and(nz, m)
      return plsc.all_reduce_population_count(nz)

    def compute(blk, rt, ra):
      oa = pl.multiple_of(blk * (RB * AL), RB * AL)
      a0 = idxa[pl.ds(oa, LANES)]
      a1 = idxa[pl.ds(oa + 16, LANES)]
      a2 = idxa[pl.ds(oa + 32, LANES)]
      la_r = (nzcount(a0) + nzcount(a1, lane < 4),
              nzcount(a1, lane >= 4) + nzcount(a2, lane < 8))

      for r in range(RB):
        b = blk * RB + r
        ot = pl.multiple_of(b * TL, TL)
        lt = zi
        for c in range(TL // LANES):
          lt = lt + nzcount(idxt[pl.ds(ot + c * LANES, LANES)])
        lt = lt + nzcount(idxt[pl.ds(ot + TL - LANES, LANES)], lane >= 8)
        la = la_r[r]

        def tstep(j, accs):
          accs = list(accs)
          for u in range(TUNROLL):
            p = j * TUNROLL + u
            m = lt > p
            for d in range(DC):
              v = rt[r * TL + p, pl.ds(d * LANES, LANES)]
              accs[d] = accs[d] + jnp.where(m, v, 0.0)
          return tuple(accs)
        acc_t = list(lax.fori_loop(0, TL // TUNROLL, tstep, (zf,) * DC))

        acc_a = [zf] * DC
        for p in range(AL):
          m = la > p
          for d in range(DC):
            v = ra[r * AL + p, pl.ds(d * LANES, LANES)]
            acc_a[d] = acc_a[d] + jnp.where(m, v, 0.0)

        inv_t = 1.0 / jnp.maximum(lt.astype(jnp.float32), 1.0)
        inv_a = 1.0 / jnp.maximum(la.astype(jnp.float32), 1.0)
        for d in range(DC):
          outb[b, pl.ds(d * LANES, LANES)] = acc_t[d] * inv_t
          outb[b, pl.ds(D + d * LANES, LANES)] = acc_a[d] * inv_a

    issue(0, rt0, ra0, sem0)

    def body(i, carry):
      blk = 2 * i
      drain(rt0, ra0, sem0)
      issue(blk + 1, rt1, ra1, sem1)
      compute(blk, rt0, ra0)
      drain(rt1, ra1, sem1)

      @pl.when(blk + 2 < NBLK)
      def _():
        issue(blk + 2, rt0, ra0, sem0)
      compute(blk + 1, rt1, ra1)
      return carry

    lax.fori_loop(0, NBLK // 2, body, 0)
    pltpu.sync_copy(outb, out_hbm.at[pl.ds(base, BPW)])

  return k(tflat, aflat, table, atable)


TCH = 8192
TSH = 12


def _tc_transpose(t_view):
  v = t_view.shape[1]
  nblk = (v + TCH - 1) // TCH

  def body(in_ref, out_ref):
    t = in_ref[...].T
    out_ref[...] = jnp.concatenate([t[:TCH // 2], t[TCH // 2:]], axis=-1)

  out = pl.pallas_call(
      body,
      grid=(nblk,),
      in_specs=[pl.BlockSpec((D, TCH), lambda g: (0, g))],
      out_specs=pl.BlockSpec((TCH // 2, 2 * D), lambda g: (g, 0)),
      out_shape=jax.ShapeDtypeStruct((nblk * TCH // 2, 2 * D), jnp.float32),
  )(t_view)
  return out.reshape(nblk * TCH, D)


def _tc_logits(feat, w, bias):
  def body(f_ref, w_ref, b_ref, o_ref):
    o_ref[...] = lax.dot_general(
        f_ref[...], w_ref[...], (((1,), (1,)), ((), ())),
        preferred_element_type=jnp.float32,
        precision=lax.Precision.HIGHEST) + b_ref[...]

  return pl.pallas_call(
      body,
      out_shape=jax.ShapeDtypeStruct((B, w.shape[0]), jnp.float32),
  )(feat, w, bias.reshape(1, -1))


def kernel(text_raw_indices, aspect_indices, embedding_matrix,
           aspect_embedding_matrix, W, b):
  tflat = text_raw_indices.astype(jnp.int32).reshape(-1)
  aflat = aspect_indices.astype(jnp.int32).reshape(-1)
  table = _tc_transpose(embedding_matrix.T)
  atable = _tc_transpose(aspect_embedding_matrix.T)
  feat = _sc_features(tflat, aflat, table, atable)
  return _tc_logits(feat, W, b)


# speedup vs baseline: 3.0152x; 1.1051x over previous
import functools

import jax
import jax.numpy as jnp
from jax import lax
from jax.experimental import pallas as pl
from jax.experimental.pallas import tpu as pltpu
from jax.experimental.pallas import tpu_sc as plsc

NC, NS, LANES = 2, 16, 16
NW = NC * NS

B, TL, AL, D = 4096, 200, 20, 64
BPW = B // NW
DC = D // LANES
RB = 2
NBLK = BPW // RB
TUNROLL = 8


def _sc_features(tflat, aflat, table, atable):
  mesh = plsc.VectorSubcoreMesh(
      core_axis_name="c", subcore_axis_name="s", num_cores=NC, num_subcores=NS)

  @functools.partial(
      pl.kernel,
      out_type=jax.ShapeDtypeStruct((B, 2 * D), jnp.float32),
      mesh=mesh,
      scratch_types=[
          pltpu.VMEM((BPW * TL,), jnp.int32),
          pltpu.VMEM((BPW * AL + LANES,), jnp.int32),
          pltpu.VMEM((RB * TL, D), jnp.float32),
          pltpu.VMEM((RB * TL, D), jnp.float32),
          pltpu.VMEM((RB * AL, D), jnp.float32),
          pltpu.VMEM((RB * AL, D), jnp.float32),
          pltpu.VMEM((BPW, 2 * D), jnp.float32),
          pltpu.SemaphoreType.DMA,
          pltpu.SemaphoreType.DMA,
      ],
      compiler_params=pltpu.CompilerParams(
          use_tc_tiling_on_sc=False, needs_layout_passes=False),
  )
  def k(tidx_hbm, aidx_hbm, tab_hbm, atab_hbm, out_hbm,
        idxt, idxa, rt0, rt1, ra0, ra1, outb, sem0, sem1):
    wid = lax.axis_index("s") * NC + lax.axis_index("c")
    base = wid * BPW
    zi = jnp.zeros((LANES,), jnp.int32)
    zf = jnp.zeros((LANES,), jnp.float32)
    lane = lax.iota(jnp.int32, LANES)

    pltpu.sync_copy(tidx_hbm.at[pl.ds(base * TL, BPW * TL)], idxt)
    pltpu.sync_copy(aidx_hbm.at[pl.ds(base * AL, BPW * AL)],
                    idxa.at[pl.ds(0, BPW * AL)])

    def _remap(v):
      return ((v & jnp.int32(-TCH)) | ((v & (TCH // 2 - 1)) << 1)
              | ((v >> TSH) & 1))

    def _rloop(buf, nwords):
      def st(j, c):
        off = pl.multiple_of(j * LANES, LANES)
        buf[pl.ds(off, LANES)] = _remap(buf[pl.ds(off, LANES)])
        return c
      lax.fori_loop(0, nwords // LANES, st, 0)

    _rloop(idxt, BPW * TL)
    _rloop(idxa, BPW * AL)

    def issue(blk, rt, ra, sem):
      ot = pl.multiple_of(blk * (RB * TL), RB * TL)
      oa = pl.multiple_of(blk * (RB * AL), RB * AL)
      pltpu.async_copy(tab_hbm.at[idxt.at[pl.ds(ot, RB * TL)]], rt, sem)
      pltpu.async_copy(atab_hbm.at[idxa.at[pl.ds(oa, RB * AL)]], ra, sem)

    def drain(rt, ra, sem):
      pltpu.make_async_copy(
          tab_hbm.at[idxt.at[pl.ds(0, RB * TL)]], rt, sem).wait()
      pltpu.make_async_copy(
          atab_hbm.at[idxa.at[pl.ds(0, RB * AL)]], ra, sem).wait()

    def nzcount(v, m=None):
      nz = v != 0
      if m is not None:
        nz = jnp.logical_and(nz, m)
      return plsc.all_reduce_population_count(nz)

    def compute(blk, rt, ra):
      oa = pl.multiple_of(blk * (RB * AL), RB * AL)
      a0 = idxa[pl.ds(oa, LANES)]
      a1 = idxa[pl.ds(oa + 16, LANES)]
      a2 = idxa[pl.ds(oa + 32, LANES)]
      la_r = (nzcount(a0) + nzcount(a1, lane < 4),
              nzcount(a1, lane >= 4) + nzcount(a2, lane < 8))

      for r in range(RB):
        b = blk * RB + r
        ot = pl.multiple_of(b * TL, TL)
        lt = zi
        for c in range(TL // LANES):
          lt = lt + nzcount(idxt[pl.ds(ot + c * LANES, LANES)])
        lt = lt + nzcount(idxt[pl.ds(ot + TL - LANES, LANES)], lane >= 8)
        la = la_r[r]

        def tstep(j, accs):
          accs = list(accs)
          for u in range(TUNROLL):
            p = j * TUNROLL + u
            m = lt > p
            for d in range(DC):
              v = rt[r * TL + p, pl.ds(d * LANES, LANES)]
              accs[d] = accs[d] + jnp.where(m, v, 0.0)
          return tuple(accs)
        acc_t = list(lax.fori_loop(0, TL // TUNROLL, tstep, (zf,) * DC))

        acc_a = [zf] * DC
        for p in range(AL):
          m = la > p
          for d in range(DC):
            v = ra[r * AL + p, pl.ds(d * LANES, LANES)]
            acc_a[d] = acc_a[d] + jnp.where(m, v, 0.0)

        inv_t = 1.0 / jnp.maximum(lt.astype(jnp.float32), 1.0)
        inv_a = 1.0 / jnp.maximum(la.astype(jnp.float32), 1.0)
        for d in range(DC):
          outb[b, pl.ds(d * LANES, LANES)] = acc_t[d] * inv_t
          outb[b, pl.ds(D + d * LANES, LANES)] = acc_a[d] * inv_a

    issue(0, rt0, ra0, sem0)

    def body(i, carry):
      blk = 2 * i
      drain(rt0, ra0, sem0)
      issue(blk + 1, rt1, ra1, sem1)
      compute(blk, rt0, ra0)
      drain(rt1, ra1, sem1)

      @pl.when(blk + 2 < NBLK)
      def _():
        issue(blk + 2, rt0, ra0, sem0)
      compute(blk + 1, rt1, ra1)
      return carry

    lax.fori_loop(0, NBLK // 2, body, 0)
    pltpu.sync_copy(outb, out_hbm.at[pl.ds(base, BPW)])

  return k(tflat, aflat, table, atable)


TCH = 16384
TSH = 13


def _tc_transpose(t_view):
  v = t_view.shape[1]
  nblk = (v + TCH - 1) // TCH

  def body(in_ref, out_ref):
    t = in_ref[...].T
    out_ref[...] = jnp.concatenate([t[:TCH // 2], t[TCH // 2:]], axis=-1)

  out = pl.pallas_call(
      body,
      grid=(nblk,),
      in_specs=[pl.BlockSpec((D, TCH), lambda g: (0, g))],
      out_specs=pl.BlockSpec((TCH // 2, 2 * D), lambda g: (g, 0)),
      out_shape=jax.ShapeDtypeStruct((nblk * TCH // 2, 2 * D), jnp.float32),
  )(t_view)
  return out.reshape(nblk * TCH, D)


def _tc_logits(feat, w, bias):
  def body(f_ref, w_ref, b_ref, o_ref):
    o_ref[...] = lax.dot_general(
        f_ref[...], w_ref[...], (((1,), (1,)), ((), ())),
        preferred_element_type=jnp.float32,
        precision=lax.Precision.HIGHEST) + b_ref[...]

  return pl.pallas_call(
      body,
      out_shape=jax.ShapeDtypeStruct((B, w.shape[0]), jnp.float32),
  )(feat, w, bias.reshape(1, -1))


def kernel(text_raw_indices, aspect_indices, embedding_matrix,
           aspect_embedding_matrix, W, b):
  tflat = text_raw_indices.astype(jnp.int32).reshape(-1)
  aflat = aspect_indices.astype(jnp.int32).reshape(-1)
  table = _tc_transpose(embedding_matrix.T)
  atable = _tc_transpose(aspect_embedding_matrix.T)
  feat = _sc_features(tflat, aflat, table, atable)
  return _tc_logits(feat, W, b)
